# baseline (speedup 1.0000x reference)
import functools

import jax
import jax.numpy as jnp
from jax import lax
from jax.experimental import pallas as pl
from jax.experimental.pallas import tpu as pltpu
from jax.experimental.pallas import tpu_sc as plsc

NUM_CORES = 2
NUM_SUBCORES = 16
LANES = 16
NW = NUM_CORES * NUM_SUBCORES

BATCH = 16384
DIM = 64
B_PER_W = BATCH // NW
CHUNK = 128
NCHUNK = B_PER_W // CHUNK


def _make_sc_kernel():
  mesh = plsc.VectorSubcoreMesh(
      core_axis_name="c", subcore_axis_name="s",
      num_cores=NUM_CORES, num_subcores=NUM_SUBCORES)

  @functools.partial(
      pl.kernel,
      out_type=jax.ShapeDtypeStruct((NW, LANES), jnp.float32),
      mesh=mesh,
      compiler_params=pltpu.CompilerParams(use_tc_tiling_on_sc=False),
      scratch_types=[
          pltpu.VMEM((NCHUNK, CHUNK), jnp.int32),
          pltpu.VMEM((NCHUNK, CHUNK), jnp.int32),
          pltpu.VMEM((NCHUNK, CHUNK), jnp.int32),
          pltpu.VMEM((B_PER_W, DIM), jnp.float32),
          pltpu.VMEM((B_PER_W, DIM), jnp.float32),
          pltpu.VMEM((B_PER_W, DIM), jnp.float32),
          pltpu.VMEM((LANES,), jnp.float32),
          pltpu.SemaphoreType.DMA,
          pltpu.SemaphoreType.DMA,
          pltpu.SemaphoreType.DMA,
      ],
  )
  def trans_e(head_hbm, tail_hbm, rel_hbm, ent_hbm, relemb_hbm, out_hbm,
              hidx, tidx, ridx, hbuf, tbuf, rbuf, accv,
              hsem, tsem, rsem):
    wid = lax.axis_index("s") * NUM_CORES + lax.axis_index("c")
    base = wid * NCHUNK

    pltpu.sync_copy(head_hbm.at[pl.ds(base, NCHUNK)], hidx)
    pltpu.sync_copy(tail_hbm.at[pl.ds(base, NCHUNK)], tidx)
    pltpu.sync_copy(rel_hbm.at[pl.ds(base, NCHUNK)], ridx)

    copies = []
    for j in range(NCHUNK):
      dst = pl.ds(j * CHUNK, CHUNK)
      copies.append(pltpu.async_copy(ent_hbm.at[hidx.at[j]], hbuf.at[dst], hsem))
      copies.append(pltpu.async_copy(ent_hbm.at[tidx.at[j]], tbuf.at[dst], tsem))
      copies.append(pltpu.async_copy(relemb_hbm.at[ridx.at[j]], rbuf.at[dst], rsem))
    for c in copies:
      c.wait()

    def body(i, accs):
      out = []
      for k in range(DIM // LANES):
        cols = pl.ds(k * LANES, LANES)
        d = hbuf[i, cols] + rbuf[i, cols] - tbuf[i, cols]
        out.append(accs[k] + d * d)
      return tuple(out)

    zero = jnp.zeros((LANES,), jnp.float32)
    accs = lax.fori_loop(0, B_PER_W, body, (zero,) * (DIM // LANES))
    accv[...] = (accs[0] + accs[1]) + (accs[2] + accs[3])
    pltpu.sync_copy(accv, out_hbm.at[wid])

  return trans_e


_trans_e_kernel = _make_sc_kernel()


def kernel(head, relation, tail, entity_emb, relation_emb):
  head2 = head.astype(jnp.int32).reshape(NW * NCHUNK, CHUNK)
  tail2 = tail.astype(jnp.int32).reshape(NW * NCHUNK, CHUNK)
  rel2 = relation.astype(jnp.int32).reshape(NW * NCHUNK, CHUNK)
  partials = _trans_e_kernel(head2, tail2, rel2, entity_emb, relation_emb)
  return jnp.sqrt(jnp.sum(partials))


# speedup vs baseline: 1.6817x; 1.6817x over previous
import functools

import jax
import jax.numpy as jnp
from jax import lax
from jax.experimental import pallas as pl
from jax.experimental.pallas import tpu as pltpu
from jax.experimental.pallas import tpu_sc as plsc

NUM_CORES = 2
NUM_SUBCORES = 16
LANES = 16
NW = NUM_CORES * NUM_SUBCORES

BATCH = 16384
DIM = 64
B_PER_W = BATCH // NW
CHUNK = 128
NCHUNK = B_PER_W // CHUNK
ROW_BYTES = DIM * 4


def _make_sc_kernel():
  mesh = plsc.VectorSubcoreMesh(
      core_axis_name="c", subcore_axis_name="s",
      num_cores=NUM_CORES, num_subcores=NUM_SUBCORES)

  @functools.partial(
      pl.kernel,
      out_type=jax.ShapeDtypeStruct((NW, LANES), jnp.float32),
      mesh=mesh,
      compiler_params=pltpu.CompilerParams(needs_layout_passes=False),
      scratch_types=[
          pltpu.VMEM((B_PER_W,), jnp.int32),
          pltpu.VMEM((B_PER_W,), jnp.int32),
          pltpu.VMEM((B_PER_W,), jnp.int32),
          pltpu.VMEM((2, CHUNK, DIM), jnp.float32),
          pltpu.VMEM((2, CHUNK, DIM), jnp.float32),
          pltpu.VMEM((2, CHUNK, DIM), jnp.float32),
          pltpu.VMEM((LANES,), jnp.float32),
          pltpu.SemaphoreType.DMA,
          pltpu.SemaphoreType.DMA,
      ],
  )
  def trans_e(head_hbm, tail_hbm, rel_idx_hbm, ent_hbm, rel_hbm, out_hbm,
              hidx, tidx, ridx, hbuf, tbuf, rbuf, accv, sem0, sem1):
    wid = lax.axis_index("s") * NUM_CORES + lax.axis_index("c")
    base = wid * B_PER_W

    pltpu.sync_copy(head_hbm.at[pl.ds(base, B_PER_W)], hidx)
    pltpu.sync_copy(tail_hbm.at[pl.ds(base, B_PER_W)], tidx)
    pltpu.sync_copy(rel_idx_hbm.at[pl.ds(base, B_PER_W)], ridx)

    sems = (sem0, sem1)

    def fire(j):
      b = j % 2
      sem = sems[b]
      off = j * CHUNK

      def enq(g, _):
        hv = hidx[pl.ds(off + g * LANES, LANES)]
        tv = tidx[pl.ds(off + g * LANES, LANES)]
        rv = ridx[pl.ds(off + g * LANES, LANES)]
        for u in range(LANES):
          i = g * LANES + u
          pltpu.async_copy(ent_hbm.at[hv[u]], hbuf.at[b, i], sem)
          pltpu.async_copy(ent_hbm.at[tv[u]], tbuf.at[b, i], sem)
          pltpu.async_copy(rel_hbm.at[rv[u]], rbuf.at[b, i], sem)
        return 0

      lax.fori_loop(0, CHUNK // LANES, enq, 0)

    def drain(j):
      b = j % 2
      sem = sems[b]

      def dr(i, _):
        pltpu.make_async_copy(ent_hbm.at[0], hbuf.at[b, 0], sem).wait()
        pltpu.make_async_copy(ent_hbm.at[0], tbuf.at[b, 0], sem).wait()
        pltpu.make_async_copy(ent_hbm.at[0], rbuf.at[b, 0], sem).wait()
        return 0

      lax.fori_loop(0, CHUNK, dr, 0)

    fire(0)
    acc = tuple(jnp.zeros((LANES,), jnp.float32) for _ in range(4))

    for j in range(NCHUNK):
      if j + 1 < NCHUNK:
        fire(j + 1)
      drain(j)
      b = j % 2

      def body(i, accs, b=b):
        out = []
        for k in range(DIM // LANES):
          cols = pl.ds(k * LANES, LANES)
          d = hbuf[b, i, cols] + rbuf[b, i, cols] - tbuf[b, i, cols]
          out.append(accs[k] + d * d)
        return tuple(out)

      acc = lax.fori_loop(0, CHUNK, body, acc)

    accv[...] = (acc[0] + acc[1]) + (acc[2] + acc[3])
    pltpu.sync_copy(accv, out_hbm.at[wid])

  return trans_e


_trans_e_kernel = _make_sc_kernel()


def kernel(head, relation, tail, entity_emb, relation_emb):
  partials = _trans_e_kernel(head.astype(jnp.int32), tail.astype(jnp.int32),
                             relation.astype(jnp.int32), entity_emb,
                             relation_emb)
  return jnp.sqrt(jnp.sum(partials))
